# baseline (speedup 1.0000x reference)
import functools

import jax
import jax.numpy as jnp
from jax import lax
from jax.experimental import pallas as pl
from jax.experimental.pallas import tpu as pltpu
from jax.experimental.pallas import tpu_sc as plsc

D = 64
H = 8
OFF = 100000
NW = 32
SCH = 512
NST = 8
PW = SCH * NST
UNROLL = 8


def _body(ids_hbm, table_hbm, out_hbm, ids_vm, gsem):
    wid = lax.axis_index("s") * 2 + lax.axis_index("c")

    offs = lax.rem(lax.iota(jnp.int32, 16), H) * OFF

    def stage(st, _):
        pltpu.sync_copy(ids_hbm.at[wid, st], ids_vm)

        def chunk(j, _):
            base = j * 16
            vec = ids_vm[pl.ds(base, 16)] + offs
            for k in range(16):
                pltpu.make_async_copy(
                    table_hbm.at[pl.ds(vec[k], 1), :],
                    out_hbm.at[wid, st, pl.ds(base + k, 1), :],
                    gsem).start()
            return 0

        lax.fori_loop(0, SCH // 16, chunk, 0)
        return 0

    lax.fori_loop(0, NST, stage, 0)

    def drain(st, _):
        pltpu.make_async_copy(table_hbm.at[pl.ds(0, SCH), :],
                              out_hbm.at[wid, st], gsem).wait()
        return 0

    lax.fori_loop(0, NST, drain, 0)


def kernel(input_ids, vocab_table):
    ids = input_ids.reshape(NW, NST, SCH)
    mesh = plsc.VectorSubcoreMesh(core_axis_name="c", subcore_axis_name="s")
    out = pl.kernel(
        _body,
        out_type=jax.ShapeDtypeStruct((NW, NST, SCH, D), jnp.float32),
        mesh=mesh,
        scratch_types=[
            pltpu.VMEM((SCH,), jnp.int32),
            pltpu.SemaphoreType.DMA,
        ],
        compiler_params=pltpu.CompilerParams(use_tc_tiling_on_sc=True),
    )(ids, vocab_table)
    return out.reshape(input_ids.shape + (D,))


# speedup vs baseline: 3.1976x; 3.1976x over previous
import functools

import jax
import jax.numpy as jnp
from jax import lax
from jax.experimental import pallas as pl
from jax.experimental.pallas import tpu as pltpu
from jax.experimental.pallas import tpu_sc as plsc

D = 64
H = 8
OFF = 100000
NW = 32
CH = 128
NCH = 32
NBUF = 6
L = 16


def _body(ids_hbm, table_hbm, out_hbm, idx_v, rows_v, gsem, osem):
    wid = lax.axis_index("s") * 2 + lax.axis_index("c")
    pltpu.sync_copy(ids_hbm.at[wid], idx_v)

    offs = lax.rem(lax.iota(jnp.int32, L), H) * OFF

    def shift(j, _):
        for k in range(CH // L):
            sl = pl.ds(k * L, L)
            idx_v[j, sl] = lax.shift_right_logical(idx_v[j, sl] + offs, 1)
        return 0

    lax.fori_loop(0, NCH, shift, 0)

    def g_copy(j, b):
        return pltpu.make_async_copy(table_hbm.at[idx_v.at[j]],
                                     rows_v.at[b], gsem.at[b])

    def o_copy(j, b):
        return pltpu.make_async_copy(rows_v.at[b], out_hbm.at[wid, j],
                                     osem.at[b])

    for b in range(NBUF):
        g_copy(b, b).start()

    def main(j, _):
        b = lax.rem(j, NBUF)
        g_copy(j, b).wait()
        o_copy(j, b).start()
        o_copy(j, b).wait()
        g_copy(j + NBUF, b).start()
        return 0

    lax.fori_loop(0, NCH - NBUF, main, 0)

    def epi(j, _):
        b = lax.rem(j, NBUF)
        g_copy(j, b).wait()
        o_copy(j, b).start()
        o_copy(j, b).wait()
        return 0

    lax.fori_loop(NCH - NBUF, NCH, epi, 0)


def kernel(input_ids, vocab_table):
    ids = input_ids.reshape(NW, NCH, CH)
    tbl = vocab_table.reshape(NW * 12500, 2 * D)
    mesh = plsc.VectorSubcoreMesh(core_axis_name="c", subcore_axis_name="s")
    pairs = pl.kernel(
        _body,
        out_type=jax.ShapeDtypeStruct((NW, NCH, CH, 2 * D), jnp.float32),
        mesh=mesh,
        scratch_types=[
            pltpu.VMEM((NCH, CH), jnp.int32),
            pltpu.VMEM((NBUF, CH, 2 * D), jnp.float32),
            pltpu.SemaphoreType.DMA((NBUF,)),
            pltpu.SemaphoreType.DMA((NBUF,)),
        ],
        compiler_params=pltpu.CompilerParams(use_tc_tiling_on_sc=True),
    )(ids, tbl)
    offsets = (jnp.arange(H, dtype=jnp.int32) * OFF).reshape(1, 1, H)
    odd = ((input_ids + offsets) & 1).astype(bool)
    pairs = pairs.reshape(input_ids.shape + (2 * D,))
    out = jnp.where(odd[..., None], pairs[..., D:], pairs[..., :D])
    return out


# speedup vs baseline: 4.2507x; 1.3294x over previous
import functools

import jax
import jax.numpy as jnp
from jax import lax
from jax.experimental import pallas as pl
from jax.experimental.pallas import tpu as pltpu
from jax.experimental.pallas import tpu_sc as plsc

D = 64
H = 8
OFF = 100000
NW = 32
CH = 128
NCH = 32
NBUF = 3
L = 16


def _body(ids_hbm, table_hbm, out_hbm, idx_v, rows_v, comp_v, gsem, osem):
    wid = lax.axis_index("s") * 2 + lax.axis_index("c")
    pltpu.sync_copy(ids_hbm.at[wid], idx_v)

    offs = lax.rem(lax.iota(jnp.int32, L), H) * OFF

    def shift(j, _):
        for k in range(CH // L):
            sl = pl.ds(k * L, L)
            idx_v[j, sl] = idx_v[j, sl] + offs
        return 0

    lax.fori_loop(0, NCH, shift, 0)

    def g_copy(j, b):
        return pltpu.make_async_copy(table_hbm.at[idx_v.at[j]],
                                     rows_v.at[b], gsem.at[b])

    def o_copy(j, b):
        return pltpu.make_async_copy(comp_v.at[b], out_hbm.at[wid, j],
                                     osem.at[b])

    def compact(b):
        def row(i, _):
            for k in range(D // L):
                comp_v[b, i, pl.ds(k * L, L)] = rows_v[b, i, pl.ds(k * L, L)]
            return 0

        lax.fori_loop(0, CH, row, 0)

    for b in range(NBUF):
        g_copy(b, b).start()

    def main(j, _):
        b = lax.rem(j, NBUF)
        g_copy(j, b).wait()
        compact(b)
        o_copy(j, b).start()
        o_copy(j, b).wait()
        g_copy(j + NBUF, b).start()
        return 0

    lax.fori_loop(0, NCH - NBUF, main, 0)

    def epi(j, _):
        b = lax.rem(j, NBUF)
        g_copy(j, b).wait()
        compact(b)
        o_copy(j, b).start()
        o_copy(j, b).wait()
        return 0

    lax.fori_loop(NCH - NBUF, NCH, epi, 0)


def kernel(input_ids, vocab_table):
    ids = input_ids.reshape(NW, NCH, CH)
    tbl = jnp.pad(vocab_table, ((0, 0), (0, 2 * D - vocab_table.shape[1])))
    mesh = plsc.VectorSubcoreMesh(core_axis_name="c", subcore_axis_name="s")
    out = pl.kernel(
        _body,
        out_type=jax.ShapeDtypeStruct((NW, NCH, CH, D), jnp.float32),
        mesh=mesh,
        scratch_types=[
            pltpu.VMEM((NCH, CH), jnp.int32),
            pltpu.VMEM((NBUF, CH, 2 * D), jnp.float32),
            pltpu.VMEM((NBUF, CH, D), jnp.float32),
            pltpu.SemaphoreType.DMA((NBUF,)),
            pltpu.SemaphoreType.DMA((NBUF,)),
        ],
        compiler_params=pltpu.CompilerParams(use_tc_tiling_on_sc=True),
    )(ids, tbl)
    return out.reshape(input_ids.shape + (D,))


# speedup vs baseline: 4.4034x; 1.0359x over previous
import functools

import jax
import jax.numpy as jnp
from jax import lax
from jax.experimental import pallas as pl
from jax.experimental.pallas import tpu as pltpu
from jax.experimental.pallas import tpu_sc as plsc

D = 64
H = 8
OFF = 100000
NW = 32
CH = 128
NCH = 32
NBUF = 5
NCB = 2
L = 16


def _body(ids_hbm, table_hbm, out_hbm, idx_v, rows_v, comp_v, gsem, osem):
    wid = lax.axis_index("s") * 2 + lax.axis_index("c")
    pltpu.sync_copy(ids_hbm.at[wid], idx_v)

    offs = lax.rem(lax.iota(jnp.int32, L), H) * OFF

    def shift(j, _):
        for k in range(CH // L):
            sl = pl.ds(k * L, L)
            idx_v[j, sl] = idx_v[j, sl] + offs
        return 0

    lax.fori_loop(0, NCH, shift, 0)

    def g_copy(j, b):
        return pltpu.make_async_copy(table_hbm.at[idx_v.at[j]],
                                     rows_v.at[b], gsem.at[b])

    def o_copy(j, c):
        return pltpu.make_async_copy(comp_v.at[c], out_hbm.at[wid, j],
                                     osem.at[c])

    def compact(b, c):
        def row(i, _):
            for k in range(D // L):
                comp_v[c, i, pl.ds(k * L, L)] = rows_v[b, i, pl.ds(k * L, L)]
            return 0

        lax.fori_loop(0, CH, row, 0)

    for b in range(NBUF):
        g_copy(b, b).start()

    def main(j, _):
        b = lax.rem(j, NBUF)
        c = lax.rem(j, NCB)
        g_copy(j, b).wait()

        @pl.when(j >= NCB)
        def _():
            o_copy(j - NCB, c).wait()

        compact(b, c)

        @pl.when(j + NBUF < NCH)
        def _():
            g_copy(j + NBUF, b).start()

        o_copy(j, c).start()
        return 0

    lax.fori_loop(0, NCH, main, 0)

    def drain(j, _):
        o_copy(j, lax.rem(j, NCB)).wait()
        return 0

    lax.fori_loop(NCH - NCB, NCH, drain, 0)


def kernel(input_ids, vocab_table):
    ids = input_ids.reshape(NW, NCH, CH)
    tbl = jnp.pad(vocab_table, ((0, 0), (0, 2 * D - vocab_table.shape[1])))
    mesh = plsc.VectorSubcoreMesh(core_axis_name="c", subcore_axis_name="s")
    out = pl.kernel(
        _body,
        out_type=jax.ShapeDtypeStruct((NW, NCH, CH, D), jnp.float32),
        mesh=mesh,
        scratch_types=[
            pltpu.VMEM((NCH, CH), jnp.int32),
            pltpu.VMEM((NBUF, CH, 2 * D), jnp.float32),
            pltpu.VMEM((NCB, CH, D), jnp.float32),
            pltpu.SemaphoreType.DMA((NBUF,)),
            pltpu.SemaphoreType.DMA((NCB,)),
        ],
        compiler_params=pltpu.CompilerParams(use_tc_tiling_on_sc=True),
    )(ids, tbl)
    return out.reshape(input_ids.shape + (D,))
